# baseline (speedup 1.0000x reference)
import jax
import jax.numpy as jnp
from jax import lax
from jax.experimental import pallas as pl
from jax.experimental.pallas import tpu as pltpu
from jax.experimental.pallas import tpu_sc as plsc

_Q = 16
_K = 4096
_D = 768
_C = 512
_CS = 32
_N_STRUCT = 17
_N_BC = 7
_NS = 16


def _body(emb_hbm, out_hbm, zidx_v, sidx_v, rows_a, rows_b, bcast_sh,
          gsem, bsem, wsem):
    cid = lax.axis_index("c")
    sid = lax.axis_index("s")
    i = sid
    part = cid
    base = i * _K
    s_end = ((i + 513 + 7) >> 3) << 3

    for q in range(2):
        zidx_v[pl.ds(q * 16, 16)] = jnp.zeros((16,), jnp.int32)
    pltpu.async_copy(emb_hbm.at[zidx_v], rows_a, gsem).wait()
    pltpu.sync_copy(
        rows_a, bcast_sh.at[pl.ds(pl.multiple_of(sid * 32, 8), 32)])
    plsc.subcore_barrier()

    @pl.when(part == 0)
    def _():
        tbufs = (rows_a, rows_b)

        def fill_idx(k):
            j0 = jnp.minimum(_CS * k, s_end - _CS)
            top = i + 512 - j0
            for q in range(_CS // 16):
                sidx_v[pl.ds(q * 16, 16)] = jnp.maximum(
                    (top - q * 16) - lax.iota(jnp.int32, 16), 0)
            return j0

        writes = []
        for k in range(_N_STRUCT):
            j0 = fill_idx(k)
            pltpu.async_copy(emb_hbm.at[sidx_v], tbufs[k % 2], gsem).wait()
            if k >= 2:
                writes[k - 2].wait()
            writes.append(
                pltpu.async_copy(
                    tbufs[k % 2],
                    out_hbm.at[pl.ds(pl.multiple_of(base + j0, 8), _CS)],
                    wsem))
        for w in writes[-2:]:
            w.wait()

    @pl.when(part == 1)
    def _():
        pend = []
        for k in range(_N_BC):
            j0 = jnp.minimum(s_end + _C * k, _K - _C)
            pend.append(
                pltpu.async_copy(
                    bcast_sh,
                    out_hbm.at[pl.ds(pl.multiple_of(base + j0, 8), _C)],
                    bsem))
        for p in pend:
            p.wait()


@jax.jit
def _rpe(emb_weight):
    mesh = plsc.VectorSubcoreMesh(core_axis_name="c", subcore_axis_name="s")
    run = pl.kernel(
        _body,
        out_type=jax.ShapeDtypeStruct((_Q * _K, _D), jnp.float32),
        mesh=mesh,
        scratch_types=[
            pltpu.VMEM((32,), jnp.int32),
            pltpu.VMEM((_CS,), jnp.int32),
            pltpu.VMEM((_CS, _D), jnp.float32),
            pltpu.VMEM((_CS, _D), jnp.float32),
            pltpu.VMEM_SHARED((_C, _D), jnp.float32),
            pltpu.SemaphoreType.DMA,
            pltpu.SemaphoreType.DMA,
            pltpu.SemaphoreType.DMA,
        ],
    )
    return run(emb_weight).reshape(_Q, _K, _D)


def kernel(q_len, k_len, emb_weight):
    return _rpe(emb_weight)


# speedup vs baseline: 1.2772x; 1.2772x over previous
import jax
import jax.numpy as jnp
from jax import lax
from jax.experimental import pallas as pl
from jax.experimental.pallas import tpu as pltpu
from jax.experimental.pallas import tpu_sc as plsc

_Q = 16
_K = 4096
_D = 768
_C = 512
_CS = 32
_N_STRUCT = 17
_N_BC = 7
_NS = 16


def _body(emb_hbm, out_hbm, zidx_v, sidx_v, rows_a, rows_b, bcast_sh,
          gsem, bsem, wsem):
    cid = lax.axis_index("c")
    sid = lax.axis_index("s")
    i = sid
    part = cid
    base = i * _K
    s_end = ((i + 513 + 7) >> 3) << 3

    for q in range(2):
        zidx_v[pl.ds(q * 16, 16)] = jnp.zeros((16,), jnp.int32)
    pltpu.async_copy(emb_hbm.at[zidx_v], rows_a, gsem).wait()
    pltpu.sync_copy(
        rows_a, bcast_sh.at[pl.ds(pl.multiple_of(sid * 32, 8), 32)])
    plsc.subcore_barrier()

    tbufs = (rows_a, rows_b)

    def fill_idx(k):
        j0 = jnp.minimum(_CS * k, s_end - _CS)
        top = i + 512 - j0
        for q in range(_CS // 16):
            sidx_v[pl.ds(q * 16, 16)] = jnp.maximum(
                (top - q * 16) - lax.iota(jnp.int32, 16), 0)
        return j0

    def do_chunk(m, k, writes):
        j0 = fill_idx(k)
        pltpu.async_copy(emb_hbm.at[sidx_v], tbufs[m % 2], gsem).wait()
        if m >= 2:
            writes[m - 2].wait()
        writes.append(
            pltpu.async_copy(
                tbufs[m % 2],
                out_hbm.at[pl.ds(pl.multiple_of(base + j0, 8), _CS)],
                wsem))

    writes = []
    for m in range(8):
        do_chunk(m, m + part * 9, writes)
    for w in writes[-2:]:
        w.wait()

    @pl.when(part == 0)
    def _():
        extra = []
        do_chunk(0, 8, extra)
        extra[0].wait()

    pend = []
    for m in range(3):
        k = m + part * 3
        j0 = jnp.minimum(s_end + _C * k, _K - _C)
        pend.append(
            pltpu.async_copy(
                bcast_sh,
                out_hbm.at[pl.ds(pl.multiple_of(base + j0, 8), _C)], bsem))

    @pl.when(part == 1)
    def _():
        j0 = jnp.minimum(s_end + _C * (_N_BC - 1), _K - _C)
        pltpu.async_copy(
            bcast_sh,
            out_hbm.at[pl.ds(pl.multiple_of(base + j0, 8), _C)],
            bsem).wait()

    for p in pend:
        p.wait()


@jax.jit
def _rpe(emb_weight):
    mesh = plsc.VectorSubcoreMesh(core_axis_name="c", subcore_axis_name="s")
    run = pl.kernel(
        _body,
        out_type=jax.ShapeDtypeStruct((_Q * _K, _D), jnp.float32),
        mesh=mesh,
        scratch_types=[
            pltpu.VMEM((32,), jnp.int32),
            pltpu.VMEM((_CS,), jnp.int32),
            pltpu.VMEM((_CS, _D), jnp.float32),
            pltpu.VMEM((_CS, _D), jnp.float32),
            pltpu.VMEM_SHARED((_C, _D), jnp.float32),
            pltpu.SemaphoreType.DMA,
            pltpu.SemaphoreType.DMA,
            pltpu.SemaphoreType.DMA,
        ],
    )
    return run(emb_weight).reshape(_Q, _K, _D)


def kernel(q_len, k_len, emb_weight):
    return _rpe(emb_weight)
